# Initial kernel scaffold; baseline (speedup 1.0000x reference)
import jax
import jax.numpy as jnp
from jax.experimental import pallas as pl


def kernel(xyz, points, W1, b1, g1, be1, W2, b2, g2, be2, W3, b3, g3, be3):
    raise NotImplementedError("write your pallas kernel here")


# speedup vs baseline: 4.4582x; 4.4582x over previous
import jax
import jax.numpy as jnp
import numpy as np
from jax import lax
from jax.experimental import pallas as pl

_NPOINT = 1024
_RADIUS = 0.2
_NSAMPLE = 32
_EPS = 1e-5

_QCHUNK = 128
_SCHUNK = 64
_SLOTS = _SCHUNK * _NSAMPLE
_NBLK = 1024


def _fps_body(xyzT_ref, far0_ref, cent_ref):
    X = xyzT_ref[:, 0, :]
    Y = xyzT_ref[:, 1, :]
    Z = xyzT_ref[:, 2, :]
    B, N = X.shape
    iota_n = lax.broadcasted_iota(jnp.int32, (B, N), 1)
    iota_s = lax.broadcasted_iota(jnp.int32, (B, _NPOINT), 1)

    def body(i, state):
        distance, far, acc = state
        acc = jnp.where(iota_s == i, far, acc)
        mask = iota_n == far
        cx = jnp.sum(jnp.where(mask, X, 0.0), axis=1, keepdims=True)
        cy = jnp.sum(jnp.where(mask, Y, 0.0), axis=1, keepdims=True)
        cz = jnp.sum(jnp.where(mask, Z, 0.0), axis=1, keepdims=True)
        dist = (X - cx) ** 2 + (Y - cy) ** 2 + (Z - cz) ** 2
        distance = jnp.minimum(distance, dist)
        m = jnp.max(distance, axis=1, keepdims=True)
        far = jnp.min(jnp.where(distance == m, iota_n, N), axis=1,
                      keepdims=True).astype(jnp.int32)
        return distance, far, acc

    distance0 = jnp.full((B, N), 1e10, dtype=jnp.float32)
    far0 = far0_ref[...]
    acc0 = jnp.zeros((B, _NPOINT), jnp.int32)
    _, _, acc = lax.fori_loop(0, _NPOINT, body, (distance0, far0, acc0))
    cent_ref[...] = acc


def _ball_body(xyzT_ref, xyz_ref, cent_ref, ball_ref, new_ref):
    Xr = xyzT_ref[0, 0, :]
    Yr = xyzT_ref[0, 1, :]
    Zr = xyzT_ref[0, 2, :]
    N = Xr.shape[0]
    q_idx = cent_ref[0, 0, :]
    oh = (lax.broadcasted_iota(jnp.int32, (N, _QCHUNK), 0)
          == q_idx[None, :]).astype(jnp.float32)
    q3 = lax.dot_general(oh, xyz_ref[0], (((0,), (0,)), ((), ())),
                         preferred_element_type=jnp.float32,
                         precision=lax.Precision.HIGHEST)
    qx = q3[:, 0:1]
    qy = q3[:, 1:2]
    qz = q3[:, 2:3]
    qq = qx * qx + qy * qy + qz * qz
    xx = Xr * Xr + Yr * Yr + Zr * Zr
    qdotx = lax.dot_general(q3, xyz_ref[0], (((1,), (1,)), ((), ())),
                            preferred_element_type=jnp.float32)
    d2 = qq + xx[None, :] - 2.0 * qdotx
    work0 = jnp.sqrt(jnp.maximum(d2, 0.0))

    iota_n = lax.broadcasted_iota(jnp.int32, (_QCHUNK, N), 1)
    iota_k = lax.broadcasted_iota(jnp.int32, (_QCHUNK, _NSAMPLE), 1)

    def body(k, state):
        work, first, acc = state
        m = jnp.min(work, axis=1, keepdims=True)
        sel = jnp.min(jnp.where(work == m, iota_n, N), axis=1,
                      keepdims=True).astype(jnp.int32)
        first = jnp.where(k == 0, sel, first)
        fin = jnp.where(m > _RADIUS, first, sel)
        acc = jnp.where(iota_k == k, fin, acc)
        work = jnp.where(iota_n == sel, jnp.inf, work)
        return work, first, acc

    first0 = jnp.zeros((_QCHUNK, 1), jnp.int32)
    acc0 = jnp.zeros((_QCHUNK, _NSAMPLE), jnp.int32)
    _, _, acc = lax.fori_loop(0, _NSAMPLE, body, (work0, first0, acc0))
    ball_ref[0] = acc
    new_ref[0] = q3


def _gather_l1_body(bf_ref, p9_ref, nq9_ref, w1t_ref, b1_ref,
                    z1_ref, st_ref):
    b = pl.program_id(0)
    c = pl.program_id(1)
    fidx = bf_ref[0, 0, :]

    def gather_step(nb, acc):
        base = nb * _NBLK
        ohT = (lax.broadcasted_iota(jnp.int32, (_NBLK, _SLOTS), 0) + base
               == fidx[None, :]).astype(jnp.float32)
        blk = p9_ref[0, pl.ds(base, _NBLK), :]
        return acc + lax.dot_general(
            ohT, blk, (((0,), (0,)), ((), ())),
            preferred_element_type=jnp.float32,
            precision=lax.Precision.HIGHEST)

    acc = lax.fori_loop(0, 8192 // _NBLK, gather_step,
                        jnp.zeros((_SLOTS, 9), jnp.float32))
    rowq = lax.broadcasted_iota(jnp.int32, (_SCHUNK, _SLOTS), 1) // _NSAMPLE
    oq = (lax.broadcasted_iota(jnp.int32, (_SCHUNK, _SLOTS), 0)
          == rowq).astype(jnp.float32)
    qexp9 = lax.dot_general(oq, nq9_ref[0], (((0,), (0,)), ((), ())),
                            preferred_element_type=jnp.float32,
                            precision=lax.Precision.HIGHEST)
    feat = acc - qexp9
    z1 = jnp.dot(feat, w1t_ref[...], preferred_element_type=jnp.float32) \
        + b1_ref[...]
    z1_ref[0, 0] = z1

    @pl.when(jnp.logical_and(b == 0, c == 0))
    def _():
        st_ref[...] = jnp.zeros_like(st_ref)

    st_ref[0:1, :] += jnp.sum(z1, axis=0, keepdims=True)
    st_ref[1:2, :] += jnp.sum(z1 * z1, axis=0, keepdims=True)


def _bn_affine(st_ref, count):
    mean = st_ref[0:1, :] / count
    ex2 = st_ref[1:2, :] / count
    var = ex2 - mean * mean
    return mean, jnp.sqrt(var + _EPS)


def _l2_body(z1_ref, st1_ref, g1_ref, be1_ref, w2t_ref, b2_ref,
             z2_ref, st_ref):
    b = pl.program_id(0)
    c = pl.program_id(1)
    count = jnp.float32(4 * _NPOINT * _NSAMPLE)
    mean, std = _bn_affine(st1_ref, count)
    z1 = z1_ref[0, 0]
    y1 = jnp.maximum((z1 - mean) / std * g1_ref[...] + be1_ref[...], 0.0)
    z2 = jnp.dot(y1, w2t_ref[...], preferred_element_type=jnp.float32) \
        + b2_ref[...]
    z2_ref[0, 0] = z2

    @pl.when(jnp.logical_and(b == 0, c == 0))
    def _():
        st_ref[...] = jnp.zeros_like(st_ref)

    st_ref[0:1, :] += jnp.sum(z2, axis=0, keepdims=True)
    st_ref[1:2, :] += jnp.sum(z2 * z2, axis=0, keepdims=True)


def _l3_body(z2_ref, st2_ref, g2_ref, be2_ref, w3t_ref, b3_ref,
             mx_ref, mn_ref, st_ref):
    b = pl.program_id(0)
    c = pl.program_id(1)
    count = jnp.float32(4 * _NPOINT * _NSAMPLE)
    mean, std = _bn_affine(st2_ref, count)
    z2 = z2_ref[0, 0]
    y2 = jnp.maximum((z2 - mean) / std * g2_ref[...] + be2_ref[...], 0.0)
    z3 = jnp.dot(y2, w3t_ref[...], preferred_element_type=jnp.float32) \
        + b3_ref[...]
    z3r = z3.reshape(_SCHUNK, _NSAMPLE, 64)
    mx_ref[0, 0] = jnp.max(z3r, axis=1)
    mn_ref[0, 0] = jnp.min(z3r, axis=1)

    @pl.when(jnp.logical_and(b == 0, c == 0))
    def _():
        st_ref[...] = jnp.zeros_like(st_ref)

    st_ref[0:1, :] += jnp.sum(z3, axis=0, keepdims=True)
    st_ref[1:2, :] += jnp.sum(z3 * z3, axis=0, keepdims=True)


def _final_body(mx_ref, mn_ref, st3_ref, g3_ref, be3_ref, out_ref):
    count = jnp.float32(4 * _NPOINT * _NSAMPLE)
    mean, std = _bn_affine(st3_ref, count)
    g3 = g3_ref[...]
    sel = jnp.where((g3 >= 0.0)[None, None, :, :], mx_ref[...], mn_ref[...])
    xh = (sel - mean[None, None, :, :]) / std[None, None, :, :]
    out_ref[...] = jnp.maximum(xh * g3[None, None, :, :]
                               + be3_ref[...][None, None, :, :], 0.0)


def kernel(xyz, points, W1, b1, g1, be1, W2, b2, g2, be2, W3, b3, g3, be3):
    B, N, _ = xyz.shape
    S, K = _NPOINT, _NSAMPLE
    f32 = jnp.float32

    xyzT = jnp.transpose(xyz, (0, 2, 1))
    far0 = jax.random.randint(jax.random.key(1), (B,), 0, N
                              ).astype(jnp.int32).reshape(B, 1)

    cent = pl.pallas_call(
        _fps_body,
        out_shape=jax.ShapeDtypeStruct((B, S), jnp.int32),
    )(xyzT, far0)

    nq = S // _QCHUNK
    cent3 = cent.reshape(B * nq, 1, _QCHUNK)
    ball, new_xyz = pl.pallas_call(
        _ball_body,
        grid=(B, nq),
        in_specs=[
            pl.BlockSpec((1, 3, N), lambda b, c: (b, 0, 0)),
            pl.BlockSpec((1, N, 3), lambda b, c: (b, 0, 0)),
            pl.BlockSpec((1, 1, _QCHUNK), lambda b, c: (b * nq + c, 0, 0)),
        ],
        out_specs=[
            pl.BlockSpec((1, _QCHUNK, K), lambda b, c: (b, c, 0)),
            pl.BlockSpec((1, _QCHUNK, 3), lambda b, c: (b, c, 0)),
        ],
        out_shape=[
            jax.ShapeDtypeStruct((B, S, K), jnp.int32),
            jax.ShapeDtypeStruct((B, S, 3), f32),
        ],
    )(xyzT, xyz, cent3)

    nc = S // _SCHUNK
    ball_flat = ball.reshape(B * nc, 1, _SLOTS)
    p9 = jnp.concatenate([xyz, points], axis=-1)
    w1t = W1.T
    nq9 = jnp.pad(new_xyz, ((0, 0), (0, 0), (0, 6)))

    z1, st1 = pl.pallas_call(
        _gather_l1_body,
        grid=(B, nc),
        in_specs=[
            pl.BlockSpec((1, 1, _SLOTS), lambda b, c: (b * nc + c, 0, 0)),
            pl.BlockSpec((1, N, 9), lambda b, c: (b, 0, 0)),
            pl.BlockSpec((1, _SCHUNK, 9), lambda b, c: (b, c, 0)),
            pl.BlockSpec((9, 32), lambda b, c: (0, 0)),
            pl.BlockSpec((1, 32), lambda b, c: (0, 0)),
        ],
        out_specs=[
            pl.BlockSpec((1, 1, _SLOTS, 32), lambda b, c: (b, c, 0, 0)),
            pl.BlockSpec((8, 32), lambda b, c: (0, 0)),
        ],
        out_shape=[
            jax.ShapeDtypeStruct((B, nc, _SLOTS, 32), f32),
            jax.ShapeDtypeStruct((8, 32), f32),
        ],
    )(ball_flat, p9, nq9, w1t, b1.reshape(1, 32))

    z2, st2 = pl.pallas_call(
        _l2_body,
        grid=(B, nc),
        in_specs=[
            pl.BlockSpec((1, 1, _SLOTS, 32), lambda b, c: (b, c, 0, 0)),
            pl.BlockSpec((8, 32), lambda b, c: (0, 0)),
            pl.BlockSpec((1, 32), lambda b, c: (0, 0)),
            pl.BlockSpec((1, 32), lambda b, c: (0, 0)),
            pl.BlockSpec((32, 32), lambda b, c: (0, 0)),
            pl.BlockSpec((1, 32), lambda b, c: (0, 0)),
        ],
        out_specs=[
            pl.BlockSpec((1, 1, _SLOTS, 32), lambda b, c: (b, c, 0, 0)),
            pl.BlockSpec((8, 32), lambda b, c: (0, 0)),
        ],
        out_shape=[
            jax.ShapeDtypeStruct((B, nc, _SLOTS, 32), f32),
            jax.ShapeDtypeStruct((8, 32), f32),
        ],
    )(z1, st1, g1.reshape(1, 32), be1.reshape(1, 32), W2.T,
      b2.reshape(1, 32))

    mx, mn, st3 = pl.pallas_call(
        _l3_body,
        grid=(B, nc),
        in_specs=[
            pl.BlockSpec((1, 1, _SLOTS, 32), lambda b, c: (b, c, 0, 0)),
            pl.BlockSpec((8, 32), lambda b, c: (0, 0)),
            pl.BlockSpec((1, 32), lambda b, c: (0, 0)),
            pl.BlockSpec((1, 32), lambda b, c: (0, 0)),
            pl.BlockSpec((32, 64), lambda b, c: (0, 0)),
            pl.BlockSpec((1, 64), lambda b, c: (0, 0)),
        ],
        out_specs=[
            pl.BlockSpec((1, 1, _SCHUNK, 64), lambda b, c: (b, c, 0, 0)),
            pl.BlockSpec((1, 1, _SCHUNK, 64), lambda b, c: (b, c, 0, 0)),
            pl.BlockSpec((8, 64), lambda b, c: (0, 0)),
        ],
        out_shape=[
            jax.ShapeDtypeStruct((B, nc, _SCHUNK, 64), f32),
            jax.ShapeDtypeStruct((B, nc, _SCHUNK, 64), f32),
            jax.ShapeDtypeStruct((8, 64), f32),
        ],
    )(z2, st2, g2.reshape(1, 32), be2.reshape(1, 32), W3.T,
      b3.reshape(1, 64))

    new_pts = pl.pallas_call(
        _final_body,
        out_shape=jax.ShapeDtypeStruct((B, nc, _SCHUNK, 64), f32),
    )(mx, mn, st3, g3.reshape(1, 64), be3.reshape(1, 64))

    return new_xyz, new_pts.reshape(B, S, 64)


# speedup vs baseline: 6.1244x; 1.3737x over previous
import jax
import jax.numpy as jnp
import numpy as np
from jax import lax
from jax.experimental import pallas as pl

_NPOINT = 1024
_RADIUS = 0.2
_NSAMPLE = 32
_EPS = 1e-5

_QCHUNK = 128
_SCHUNK = 64
_SLOTS = _SCHUNK * _NSAMPLE
_NBLK = 1024


def _split3(x):
    hi = x.astype(jnp.bfloat16)
    r = x - hi.astype(jnp.float32)
    mid = r.astype(jnp.bfloat16)
    lo = (r - mid.astype(jnp.float32)).astype(jnp.bfloat16)
    return hi, mid, lo


def _onehot_gatherT(ohT_bf16, vals_f32, dims):
    hi, mid, lo = _split3(vals_f32)
    out = lax.dot_general(ohT_bf16, hi, dims,
                          preferred_element_type=jnp.float32)
    out = out + lax.dot_general(ohT_bf16, mid, dims,
                                preferred_element_type=jnp.float32)
    return out + lax.dot_general(ohT_bf16, lo, dims,
                                 preferred_element_type=jnp.float32)


def _fps_body(xyzT_ref, far0_ref, cent_ref):
    X = xyzT_ref[:, 0, :]
    Y = xyzT_ref[:, 1, :]
    Z = xyzT_ref[:, 2, :]
    B, N = X.shape
    iota_n = lax.broadcasted_iota(jnp.int32, (B, N), 1)
    iota_s = lax.broadcasted_iota(jnp.int32, (B, _NPOINT), 1)

    def body(i, state):
        distance, far, acc = state
        acc = jnp.where(iota_s == i, far, acc)
        mask = iota_n == far
        cx = jnp.sum(jnp.where(mask, X, 0.0), axis=1, keepdims=True)
        cy = jnp.sum(jnp.where(mask, Y, 0.0), axis=1, keepdims=True)
        cz = jnp.sum(jnp.where(mask, Z, 0.0), axis=1, keepdims=True)
        dist = (X - cx) ** 2 + (Y - cy) ** 2 + (Z - cz) ** 2
        distance = jnp.minimum(distance, dist)
        m = jnp.max(distance, axis=1, keepdims=True)
        far = jnp.min(jnp.where(distance == m, iota_n, N), axis=1,
                      keepdims=True).astype(jnp.int32)
        return distance, far, acc

    distance0 = jnp.full((B, N), 1e10, dtype=jnp.float32)
    far0 = far0_ref[...]
    acc0 = jnp.zeros((B, _NPOINT), jnp.int32)
    _, _, acc = lax.fori_loop(0, _NPOINT, body, (distance0, far0, acc0))
    cent_ref[...] = acc


def _ball_body(xyzT_ref, xyz_ref, cent_ref, ball_ref, new_ref):
    Xr = xyzT_ref[0, 0, :]
    Yr = xyzT_ref[0, 1, :]
    Zr = xyzT_ref[0, 2, :]
    N = Xr.shape[0]
    q_idx = cent_ref[0, 0, :]
    oh = (lax.broadcasted_iota(jnp.int32, (N, _QCHUNK), 0)
          == q_idx[None, :]).astype(jnp.bfloat16)
    q3 = _onehot_gatherT(oh, xyz_ref[0], (((0,), (0,)), ((), ())))
    qx = q3[:, 0:1]
    qy = q3[:, 1:2]
    qz = q3[:, 2:3]
    qq = qx * qx + qy * qy + qz * qz
    xx = Xr * Xr + Yr * Yr + Zr * Zr
    qdotx = lax.dot_general(q3, xyz_ref[0], (((1,), (1,)), ((), ())),
                            preferred_element_type=jnp.float32)
    d2 = qq + xx[None, :] - 2.0 * qdotx
    work0 = jnp.sqrt(jnp.maximum(d2, 0.0))

    iota_n = lax.broadcasted_iota(jnp.int32, (_QCHUNK, N), 1)
    iota_k = lax.broadcasted_iota(jnp.int32, (_QCHUNK, _NSAMPLE), 1)

    def body(k, state):
        work, first, acc = state
        m = jnp.min(work, axis=1, keepdims=True)
        sel = jnp.min(jnp.where(work == m, iota_n, N), axis=1,
                      keepdims=True).astype(jnp.int32)
        first = jnp.where(k == 0, sel, first)
        fin = jnp.where(m > _RADIUS, first, sel)
        acc = jnp.where(iota_k == k, fin, acc)
        work = jnp.where(iota_n == sel, jnp.inf, work)
        return work, first, acc

    first0 = jnp.zeros((_QCHUNK, 1), jnp.int32)
    acc0 = jnp.zeros((_QCHUNK, _NSAMPLE), jnp.int32)
    _, _, acc = lax.fori_loop(0, _NSAMPLE, body, (work0, first0, acc0))
    ball_ref[0] = acc
    new_ref[0] = q3


def _gather_l1_body(bf_ref, p9_ref, nq9_ref, w1t_ref, b1_ref,
                    z1_ref, st_ref):
    b = pl.program_id(0)
    c = pl.program_id(1)
    fidx = bf_ref[0, 0, :]

    def gather_step(nb, acc):
        base = nb * _NBLK
        ohT = (lax.broadcasted_iota(jnp.int32, (_NBLK, _SLOTS), 0) + base
               == fidx[None, :]).astype(jnp.bfloat16)
        blk = p9_ref[0, pl.ds(base, _NBLK), :]
        return acc + _onehot_gatherT(ohT, blk, (((0,), (0,)), ((), ())))

    acc = lax.fori_loop(0, 8192 // _NBLK, gather_step,
                        jnp.zeros((_SLOTS, 9), jnp.float32))
    rowq = lax.broadcasted_iota(jnp.int32, (_SCHUNK, _SLOTS), 1) // _NSAMPLE
    oq = (lax.broadcasted_iota(jnp.int32, (_SCHUNK, _SLOTS), 0)
          == rowq).astype(jnp.float32)
    qexp9 = lax.dot_general(oq, nq9_ref[0], (((0,), (0,)), ((), ())),
                            preferred_element_type=jnp.float32,
                            precision=lax.Precision.HIGHEST)
    feat = acc - qexp9
    z1 = jnp.dot(feat, w1t_ref[...], preferred_element_type=jnp.float32) \
        + b1_ref[...]
    z1_ref[0, 0] = z1

    @pl.when(jnp.logical_and(b == 0, c == 0))
    def _():
        st_ref[...] = jnp.zeros_like(st_ref)

    st_ref[0:1, :] += jnp.sum(z1, axis=0, keepdims=True)
    st_ref[1:2, :] += jnp.sum(z1 * z1, axis=0, keepdims=True)


def _bn_affine(st_ref, count):
    mean = st_ref[0:1, :] / count
    ex2 = st_ref[1:2, :] / count
    var = ex2 - mean * mean
    return mean, jnp.sqrt(var + _EPS)


def _l2_body(z1_ref, st1_ref, g1_ref, be1_ref, w2t_ref, b2_ref,
             z2_ref, st_ref):
    b = pl.program_id(0)
    c = pl.program_id(1)
    count = jnp.float32(4 * _NPOINT * _NSAMPLE)
    mean, std = _bn_affine(st1_ref, count)
    z1 = z1_ref[0, 0]
    y1 = jnp.maximum((z1 - mean) / std * g1_ref[...] + be1_ref[...], 0.0)
    z2 = jnp.dot(y1, w2t_ref[...], preferred_element_type=jnp.float32) \
        + b2_ref[...]
    z2_ref[0, 0] = z2

    @pl.when(jnp.logical_and(b == 0, c == 0))
    def _():
        st_ref[...] = jnp.zeros_like(st_ref)

    st_ref[0:1, :] += jnp.sum(z2, axis=0, keepdims=True)
    st_ref[1:2, :] += jnp.sum(z2 * z2, axis=0, keepdims=True)


def _l3_body(z2_ref, st2_ref, g2_ref, be2_ref, w3t_ref, b3_ref,
             mx_ref, mn_ref, st_ref):
    b = pl.program_id(0)
    c = pl.program_id(1)
    count = jnp.float32(4 * _NPOINT * _NSAMPLE)
    mean, std = _bn_affine(st2_ref, count)
    z2 = z2_ref[0, 0]
    y2 = jnp.maximum((z2 - mean) / std * g2_ref[...] + be2_ref[...], 0.0)
    z3 = jnp.dot(y2, w3t_ref[...], preferred_element_type=jnp.float32) \
        + b3_ref[...]
    z3r = z3.reshape(_SCHUNK, _NSAMPLE, 64)
    mx_ref[0, 0] = jnp.max(z3r, axis=1)
    mn_ref[0, 0] = jnp.min(z3r, axis=1)

    @pl.when(jnp.logical_and(b == 0, c == 0))
    def _():
        st_ref[...] = jnp.zeros_like(st_ref)

    st_ref[0:1, :] += jnp.sum(z3, axis=0, keepdims=True)
    st_ref[1:2, :] += jnp.sum(z3 * z3, axis=0, keepdims=True)


def _final_body(mx_ref, mn_ref, st3_ref, g3_ref, be3_ref, out_ref):
    count = jnp.float32(4 * _NPOINT * _NSAMPLE)
    mean, std = _bn_affine(st3_ref, count)
    g3 = g3_ref[...]
    sel = jnp.where((g3 >= 0.0)[None, None, :, :], mx_ref[...], mn_ref[...])
    xh = (sel - mean[None, None, :, :]) / std[None, None, :, :]
    out_ref[...] = jnp.maximum(xh * g3[None, None, :, :]
                               + be3_ref[...][None, None, :, :], 0.0)


def kernel(xyz, points, W1, b1, g1, be1, W2, b2, g2, be2, W3, b3, g3, be3):
    B, N, _ = xyz.shape
    S, K = _NPOINT, _NSAMPLE
    f32 = jnp.float32

    xyzT = jnp.transpose(xyz, (0, 2, 1))
    far0 = jax.random.randint(jax.random.key(1), (B,), 0, N
                              ).astype(jnp.int32).reshape(B, 1)

    cent = pl.pallas_call(
        _fps_body,
        out_shape=jax.ShapeDtypeStruct((B, S), jnp.int32),
    )(xyzT, far0)

    nq = S // _QCHUNK
    cent3 = cent.reshape(B * nq, 1, _QCHUNK)
    ball, new_xyz = pl.pallas_call(
        _ball_body,
        grid=(B, nq),
        in_specs=[
            pl.BlockSpec((1, 3, N), lambda b, c: (b, 0, 0)),
            pl.BlockSpec((1, N, 3), lambda b, c: (b, 0, 0)),
            pl.BlockSpec((1, 1, _QCHUNK), lambda b, c: (b * nq + c, 0, 0)),
        ],
        out_specs=[
            pl.BlockSpec((1, _QCHUNK, K), lambda b, c: (b, c, 0)),
            pl.BlockSpec((1, _QCHUNK, 3), lambda b, c: (b, c, 0)),
        ],
        out_shape=[
            jax.ShapeDtypeStruct((B, S, K), jnp.int32),
            jax.ShapeDtypeStruct((B, S, 3), f32),
        ],
    )(xyzT, xyz, cent3)

    nc = S // _SCHUNK
    ball_flat = ball.reshape(B * nc, 1, _SLOTS)
    p9 = jnp.concatenate([xyz, points], axis=-1)
    w1t = W1.T
    nq9 = jnp.pad(new_xyz, ((0, 0), (0, 0), (0, 6)))

    z1, st1 = pl.pallas_call(
        _gather_l1_body,
        grid=(B, nc),
        in_specs=[
            pl.BlockSpec((1, 1, _SLOTS), lambda b, c: (b * nc + c, 0, 0)),
            pl.BlockSpec((1, N, 9), lambda b, c: (b, 0, 0)),
            pl.BlockSpec((1, _SCHUNK, 9), lambda b, c: (b, c, 0)),
            pl.BlockSpec((9, 32), lambda b, c: (0, 0)),
            pl.BlockSpec((1, 32), lambda b, c: (0, 0)),
        ],
        out_specs=[
            pl.BlockSpec((1, 1, _SLOTS, 32), lambda b, c: (b, c, 0, 0)),
            pl.BlockSpec((8, 32), lambda b, c: (0, 0)),
        ],
        out_shape=[
            jax.ShapeDtypeStruct((B, nc, _SLOTS, 32), f32),
            jax.ShapeDtypeStruct((8, 32), f32),
        ],
    )(ball_flat, p9, nq9, w1t, b1.reshape(1, 32))

    z2, st2 = pl.pallas_call(
        _l2_body,
        grid=(B, nc),
        in_specs=[
            pl.BlockSpec((1, 1, _SLOTS, 32), lambda b, c: (b, c, 0, 0)),
            pl.BlockSpec((8, 32), lambda b, c: (0, 0)),
            pl.BlockSpec((1, 32), lambda b, c: (0, 0)),
            pl.BlockSpec((1, 32), lambda b, c: (0, 0)),
            pl.BlockSpec((32, 32), lambda b, c: (0, 0)),
            pl.BlockSpec((1, 32), lambda b, c: (0, 0)),
        ],
        out_specs=[
            pl.BlockSpec((1, 1, _SLOTS, 32), lambda b, c: (b, c, 0, 0)),
            pl.BlockSpec((8, 32), lambda b, c: (0, 0)),
        ],
        out_shape=[
            jax.ShapeDtypeStruct((B, nc, _SLOTS, 32), f32),
            jax.ShapeDtypeStruct((8, 32), f32),
        ],
    )(z1, st1, g1.reshape(1, 32), be1.reshape(1, 32), W2.T,
      b2.reshape(1, 32))

    mx, mn, st3 = pl.pallas_call(
        _l3_body,
        grid=(B, nc),
        in_specs=[
            pl.BlockSpec((1, 1, _SLOTS, 32), lambda b, c: (b, c, 0, 0)),
            pl.BlockSpec((8, 32), lambda b, c: (0, 0)),
            pl.BlockSpec((1, 32), lambda b, c: (0, 0)),
            pl.BlockSpec((1, 32), lambda b, c: (0, 0)),
            pl.BlockSpec((32, 64), lambda b, c: (0, 0)),
            pl.BlockSpec((1, 64), lambda b, c: (0, 0)),
        ],
        out_specs=[
            pl.BlockSpec((1, 1, _SCHUNK, 64), lambda b, c: (b, c, 0, 0)),
            pl.BlockSpec((1, 1, _SCHUNK, 64), lambda b, c: (b, c, 0, 0)),
            pl.BlockSpec((8, 64), lambda b, c: (0, 0)),
        ],
        out_shape=[
            jax.ShapeDtypeStruct((B, nc, _SCHUNK, 64), f32),
            jax.ShapeDtypeStruct((B, nc, _SCHUNK, 64), f32),
            jax.ShapeDtypeStruct((8, 64), f32),
        ],
    )(z2, st2, g2.reshape(1, 32), be2.reshape(1, 32), W3.T,
      b3.reshape(1, 64))

    new_pts = pl.pallas_call(
        _final_body,
        out_shape=jax.ShapeDtypeStruct((B, nc, _SCHUNK, 64), f32),
    )(mx, mn, st3, g3.reshape(1, 64), be3.reshape(1, 64))

    return new_xyz, new_pts.reshape(B, S, 64)
